# baseline (speedup 1.0000x reference)
import jax
import jax.numpy as jnp
from jax import lax
from jax.experimental import pallas as pl
from jax.experimental.pallas import tpu as pltpu
from jax.experimental.pallas import tpu_sc as plsc

N = 10000
E = 160000
B = 1024
NS = 16
NC = 2
L = 16
TILES = NC * NS
SLOTS_T = B // TILES
E_TILE = E // NS
CH = 800
CH_DEG = 400
NCH_DEG = E_TILE // CH_DEG
NCH_ALL = E // CH
CAP = 3584
FLUSH_T = CAP - CH - 2 * L
DD = 1024
DP = 2812
DPH = 1408
NPAD = 10240
SEG = NPAD // NS

f32 = jnp.float32
i32 = jnp.int32


def _leaky(x):
    return jnp.where(x > 0, x, 0.01 * x)


def _bn(x, gamma, beta, eps=1e-5):
    m = jnp.mean(x, axis=0, keepdims=True)
    v = jnp.mean((x - m) * (x - m), axis=0, keepdims=True)
    return (x - m) / jnp.sqrt(v + eps) * gamma + beta


def _frsqrt(x):
    i = plsc.bitcast(x, i32)
    y = plsc.bitcast(jnp.int32(0x5F3759DF) - (i >> 1), f32)
    for _ in range(3):
        y = y * (1.5 - 0.5 * x * y * y)
    return y


def _sc_body(d_row, d_col, d_ew, d_idx, p_row, p_col, p_ew, p_idx, xd, xp,
             agg, repd, repp,
             idx_tab, rep_tab, dinv, degtmp,
             row_buf, col_buf, ew_buf,
             slot_list, row_list, norm_list, idx16a, idx16b,
             stage_d, stage_p, acc, deg_scr, deg_sum, sem):
    c = lax.axis_index("c")
    s = lax.axis_index("s")
    t = c * NS + s
    ii = lax.iota(i32, L)
    zf = jnp.zeros((L,), f32)
    zi = jnp.zeros((L,), i32)

    def make_dinv(col_hbm, ew_hbm):
        def zv(v, _):
            dinv[pl.ds(v * L, L)] = zf
            return _
        lax.fori_loop(0, NPAD // L, zv, None)
        def chunk(ch, _):
            base = s * E_TILE + ch * CH_DEG
            pltpu.sync_copy(col_hbm.at[pl.ds(base, CH_DEG)],
                            col_buf.at[pl.ds(0, CH_DEG)])
            pltpu.sync_copy(ew_hbm.at[pl.ds(base, CH_DEG)],
                            ew_buf.at[pl.ds(0, CH_DEG)])
            def eb(e, _):
                cv = col_buf[pl.ds(e * L, L)]
                ev = ew_buf[pl.ds(e * L, L)]
                def db(rem):
                    plsc.store_scatter(rep_tab, [cv], ii, mask=rem)
                    win = (plsc.load_gather(rep_tab, [cv]) == ii) & rem
                    plsc.addupdate_scatter(dinv, [cv], ev, mask=win)
                    return rem & jnp.logical_not(win)
                lax.while_loop(jnp.any, db, cv == cv)
                return _
            lax.fori_loop(0, CH_DEG // L, eb, None)
            return _
        lax.fori_loop(0, NCH_DEG, chunk, None)
        def wr(k, _):
            pltpu.sync_copy(dinv.at[pl.ds(k * SEG, SEG)],
                            deg_scr.at[pl.ds(s * NPAD + k * SEG, SEG)])
            return _
        lax.fori_loop(0, NS, wr, None)
        plsc.subcore_barrier()
        def zv2(v, _):
            dinv[pl.ds(v * L, L)] = zf
            return _
        lax.fori_loop(0, SEG // L, zv2, None)
        def tsum(tt, _):
            pltpu.sync_copy(deg_scr.at[pl.ds(tt * NPAD + s * SEG, SEG)],
                            degtmp.at[pl.ds(0, SEG)])
            def vadd(v, _):
                dinv[pl.ds(v * L, L)] = (dinv[pl.ds(v * L, L)]
                                         + degtmp[pl.ds(v * L, L)])
                return _
            lax.fori_loop(0, SEG // L, vadd, None, unroll=8)
            return _
        lax.fori_loop(0, NS, tsum, None)
        pltpu.sync_copy(dinv.at[pl.ds(0, SEG)],
                        deg_sum.at[pl.ds(s * SEG, SEG)])
        plsc.subcore_barrier()
        def rd(k, _):
            pltpu.sync_copy(deg_sum.at[pl.ds(k * SEG, SEG)],
                            dinv.at[pl.ds(k * SEG, SEG)])
            return _
        lax.fori_loop(0, NS, rd, None)
        def nb(i, _):
            x = dinv[pl.ds(i * L, L)]
            dinv[pl.ds(i * L, L)] = _frsqrt(x + 1.0)
            return _
        lax.fori_loop(0, N // L, nb, None)

    def build_rep(idx_hbm):
        def ld(k, _):
            pltpu.sync_copy(idx_hbm.at[pl.ds(k * 512, 512)],
                            idx_tab.at[pl.ds(k * 512, 512)])
            return _
        lax.fori_loop(0, B // 512, ld, None)
        def zr(v, _):
            rep_tab[pl.ds(v * L, L)] = zi
            return _
        lax.fori_loop(0, N // L, zr, None)
        def br(j, _):
            cv = idx_tab[pl.ds(j * L, L)]
            plsc.store_scatter(rep_tab, [cv], ii + j * L)
            return _
        lax.fori_loop(0, B // L, br, None)

    def edge_flush(off, gather_fn, W, stage):
        ng = (off + L - 1) // L
        def g(k, _):
            idx16a[...] = row_list[pl.ds(k * L, L)]
            slotv = slot_list[pl.ds(k * L, L)] - t * SLOTS_T
            nv = norm_list[pl.ds(k * L, L)]
            gather_fn(idx16a, stage)
            for r in range(L):
                nsc = jnp.sum(jnp.where(ii == r, nv, 0.0))
                sl = jnp.sum(jnp.where(ii == r, slotv, 0))
                def fb(f, _):
                    acc[sl, pl.ds(f * L, L)] = (
                        acc[sl, pl.ds(f * L, L)]
                        + stage[r, pl.ds(f * L, L)] * nsc)
                    return _
                lax.fori_loop(0, W // L, fb, None, unroll=4)
            return _
        lax.fori_loop(0, ng, g, None)

    def pad_lists(off):
        slot_list[pl.ds(off, L)] = zi + (t * SLOTS_T + SLOTS_T)
        row_list[pl.ds(off, L)] = zi
        norm_list[pl.ds(off, L)] = zf

    def run_pass(row_hbm, col_hbm, ew_hbm, gather_fn, store_fn, W, stage):
        for rr in range(SLOTS_T + 1):
            def za(f, _):
                acc[rr, pl.ds(f * L, L)] = zf
                return _
            lax.fori_loop(0, W // L, za, None, unroll=4)

        def edge_scan(off):
            def vb(v, off):
                cv = col_buf[pl.ds(v * L, L)]
                sv = plsc.load_gather(rep_tab, [cv])
                may = (sv >> 5) == t
                def full(o):
                    rv = row_buf[pl.ds(v * L, L)]
                    ev = ew_buf[pl.ds(v * L, L)]
                    chk = plsc.load_gather(idx_tab, [sv])
                    valid = may & (chk == cv)
                    dr = plsc.load_gather(dinv, [rv])
                    dc = plsc.load_gather(dinv, [cv])
                    nv = dr * ev * dc
                    plsc.store_compressed(slot_list.at[pl.ds(o, L)], sv,
                                          mask=valid)
                    plsc.store_compressed(row_list.at[pl.ds(o, L)], rv,
                                          mask=valid)
                    plsc.store_compressed(norm_list.at[pl.ds(o, L)], nv,
                                          mask=valid)
                    return o + jnp.sum(jnp.where(valid, 1, 0))
                return lax.cond(jnp.any(may), full, lambda o: o, off)
            return lax.fori_loop(0, CH // L, vb, off)

        def self_scan(off):
            def jb(j, off):
                bv = ii + j * L
                cv = idx_tab[pl.ds(j * L, L)]
                rv = plsc.load_gather(rep_tab, [cv])
                valid = (rv == bv) & ((bv >> 5) == t)
                dc = plsc.load_gather(dinv, [cv])
                nv = dc * dc
                plsc.store_compressed(slot_list.at[pl.ds(off, L)], bv,
                                      mask=valid)
                plsc.store_compressed(row_list.at[pl.ds(off, L)], cv,
                                      mask=valid)
                plsc.store_compressed(norm_list.at[pl.ds(off, L)], nv,
                                      mask=valid)
                return off + jnp.sum(jnp.where(valid, 1, 0))
            return lax.fori_loop(0, B // L, jb, off)

        def chunk(ch, off):
            is_tail = ch == NCH_ALL
            @pl.when(jnp.logical_not(is_tail))
            def _():
                base = ch * CH
                pltpu.sync_copy(row_hbm.at[pl.ds(base, CH)],
                                row_buf.at[pl.ds(0, CH)])
                pltpu.sync_copy(col_hbm.at[pl.ds(base, CH)],
                                col_buf.at[pl.ds(0, CH)])
                pltpu.sync_copy(ew_hbm.at[pl.ds(base, CH)],
                                ew_buf.at[pl.ds(0, CH)])
            off = lax.cond(is_tail, self_scan, edge_scan, off)
            def do_flush(o):
                pad_lists(o)
                edge_flush(o, gather_fn, W, stage)
                return jnp.int32(0)
            return lax.cond((off >= FLUSH_T) | is_tail, do_flush,
                            lambda o: o, off)
        lax.fori_loop(0, NCH_ALL + 1, chunk, jnp.int32(0))

        for g in range(SLOTS_T // L):
            for r in range(L):
                def mf(f, _):
                    stage[r, pl.ds(f * L, L)] = acc[g * L + r, pl.ds(f * L, L)]
                    return _
                lax.fori_loop(0, W // L, mf, None, unroll=4)
            store_fn(g, stage)

    def export_rep(rep_out):
        @pl.when(c == 0)
        def _():
            def jb(j, _):
                cv = idx_tab[pl.ds(s * (B // NS) + j * L, L)]
                rv = plsc.load_gather(rep_tab, [cv])
                row_buf[pl.ds(j * L, L)] = rv
                return _
            lax.fori_loop(0, (B // NS) // L, jb, None)
            pltpu.sync_copy(row_buf.at[pl.ds(0, B // NS)],
                            rep_out.at[pl.ds(s * (B // NS), B // NS)])

    make_dinv(d_col, d_ew)
    build_rep(d_idx)
    def d_gather(idx_ref, stage):
        pltpu.async_copy(xd.at[idx_ref], stage, sem).wait()
    def d_store(g, stage):
        row0 = pl.multiple_of(t * SLOTS_T + g * L, 8)
        pltpu.sync_copy(stage, agg.at[pl.ds(row0, L), pl.ds(0, DD)])
    run_pass(d_row, d_col, d_ew, d_gather, d_store, DD, stage_d)
    export_rep(repd)

    plsc.subcore_barrier()
    make_dinv(p_col, p_ew)
    build_rep(p_idx)
    def ph(h, _):
        col0 = pl.multiple_of(h * DPH, 128)
        def p_gather(idx_ref, stage):
            pltpu.async_copy(xp.at[idx_ref, pl.ds(col0, DPH)], stage,
                             sem).wait()
        def p_store(g, stage):
            row0 = pl.multiple_of(t * SLOTS_T + g * L, 8)
            colb = pl.multiple_of(DD + col0, 128)
            pltpu.sync_copy(stage, agg.at[pl.ds(row0, L), pl.ds(colb, DPH)])
        run_pass(p_row, p_col, p_ew, p_gather, p_store, DPH, stage_p)
        return _
    lax.fori_loop(0, 2, ph, None)
    export_rep(repp)


def _sc_aggregate(d_row, d_col, d_ew, d_idx, p_row, p_col, p_ew, p_idx, xd, xp):
    mesh = plsc.VectorSubcoreMesh(core_axis_name="c", subcore_axis_name="s")
    fn = pl.kernel(
        _sc_body,
        out_type=(
            jax.ShapeDtypeStruct((B, DD + 2 * DPH), f32),
            jax.ShapeDtypeStruct((B,), i32),
            jax.ShapeDtypeStruct((B,), i32),
        ),
        mesh=mesh,
        compiler_params=pltpu.CompilerParams(needs_layout_passes=False),
        scratch_types=[
            pltpu.VMEM((B,), i32),
            pltpu.VMEM((N,), i32),
            pltpu.VMEM((NPAD,), f32),
            pltpu.VMEM((SEG,), f32),
            pltpu.VMEM((CH + L,), i32),
            pltpu.VMEM((CH + L,), i32),
            pltpu.VMEM((CH + L,), f32),
            pltpu.VMEM((CAP,), i32),
            pltpu.VMEM((CAP,), i32),
            pltpu.VMEM((CAP,), f32),
            pltpu.VMEM((L,), i32),
            pltpu.VMEM((L,), i32),
            pltpu.VMEM((L, DD), f32),
            pltpu.VMEM((L, DPH), f32),
            pltpu.VMEM((SLOTS_T + 1, DPH), f32),
            pltpu.HBM((NS * NPAD,), f32),
            pltpu.HBM((NPAD,), f32),
            pltpu.SemaphoreType.DMA,
        ],
    )
    return fn(d_row, d_col, d_ew, d_idx, p_row, p_col, p_ew, p_idx, xd, xp)


def _tc_encode(agg, repd, repp, Wd, bd, Wp, bp):
    def body(ag_ref, rd_ref, rp_ref, wd_ref, bd_ref, wp_ref, bp_ref,
             ec_ref, go_ref):
        iota2 = lax.broadcasted_iota(i32, (B, B), 1)
        pd = (rd_ref[...] == iota2).astype(f32)
        pp = (rp_ref[...] == iota2).astype(f32)
        ag = ag_ref[...]
        rd = jnp.dot(pd, ag[:, :DD], preferred_element_type=f32)
        rp = jnp.dot(pp, ag[:, DD:DD + DP], preferred_element_type=f32)
        ec = jnp.dot(rd, wd_ref[...], preferred_element_type=f32)
        ec_ref[...] = _leaky(ec + bd_ref[...])
        go = jnp.dot(rp, wp_ref[...], preferred_element_type=f32)
        go_ref[...] = _leaky(go + bp_ref[...])
    return pl.pallas_call(
        body,
        compiler_params=pltpu.CompilerParams(
            vmem_limit_bytes=100 * 1024 * 1024),
        out_shape=(
            jax.ShapeDtypeStruct((B, 1024), f32),
            jax.ShapeDtypeStruct((B, 1024), f32),
        ),
    )(agg, repd.reshape(B, 1), repp.reshape(B, 1),
      Wd, bd.reshape(1, -1), Wp, bp.reshape(1, -1))


def _tc_mlp(dv, pe, ec, go, W1a, W1b, W1c, W1d, b1, g1, be1,
            W2, b2, g2, be2, W3, b3, g3, be3, W4, b4):
    def body(dv_ref, pe_ref, ec_ref, go_ref, w1a_ref, w1b_ref, w1c_ref,
             w1d_ref, b1_ref, g1_ref, be1_ref, w2_ref, b2_ref, g2_ref,
             be2_ref, w3_ref, b3_ref, g3_ref, be3_ref, w4_ref, b4_ref,
             out_ref, feat_ref):
        h = (jnp.dot(dv_ref[...], w1a_ref[...], preferred_element_type=f32)
             + jnp.dot(pe_ref[...], w1b_ref[...], preferred_element_type=f32)
             + jnp.dot(ec_ref[...], w1c_ref[...], preferred_element_type=f32)
             + jnp.dot(go_ref[...], w1d_ref[...], preferred_element_type=f32)
             + b1_ref[...])
        h = _leaky(_bn(h, g1_ref[...], be1_ref[...]))
        feat = _leaky(_bn(
            jnp.dot(h, w2_ref[...], preferred_element_type=f32) + b2_ref[...],
            g2_ref[...], be2_ref[...]))
        feat_ref[...] = feat
        z = _bn(_leaky(
            jnp.dot(feat, w3_ref[...], preferred_element_type=f32) + b3_ref[...]),
            g3_ref[...], be3_ref[...])
        out_ref[...] = jnp.dot(z, w4_ref[...], preferred_element_type=f32) + b4_ref[...]
    r2 = lambda a: a.reshape(1, -1)
    return pl.pallas_call(
        body,
        compiler_params=pltpu.CompilerParams(
            vmem_limit_bytes=100 * 1024 * 1024),
        out_shape=(
            jax.ShapeDtypeStruct((B, 1), f32),
            jax.ShapeDtypeStruct((B, 1024), f32),
        ),
    )(dv, pe, ec, go, W1a, W1b, W1c, W1d, r2(b1), r2(g1), r2(be1),
      W2, r2(b2), r2(g2), r2(be2), W3, r2(b3), r2(g3), r2(be3), W4, r2(b4))


def kernel(d_index, p_index, d_vecs, p_embeddings, y, d_ecfps, d_edge_index,
           d_edge_weight, p_gos, p_edge_index, p_edge_weight, Wd, bd, Wp, bp,
           W1, b1, g1, be1, W2, b2, g2, be2, W3, b3, g3, be3, W4, b4):
    d_row = d_edge_index[0].astype(i32)
    d_col = d_edge_index[1].astype(i32)
    p_row = p_edge_index[0].astype(i32)
    p_col = p_edge_index[1].astype(i32)
    d_idx = d_index.astype(i32)
    p_idx = p_index.astype(i32)

    xp_pad = jnp.pad(p_gos, ((0, 0), (0, 2 * DPH - DP)))
    agg, repd, repp = _sc_aggregate(
        d_row, d_col, d_edge_weight, d_idx,
        p_row, p_col, p_edge_weight, p_idx, d_ecfps, xp_pad)
    ec, go = _tc_encode(agg, repd, repp, Wd, bd, Wp, bp)
    W1a = W1[:300]
    W1b = W1[300:1324]
    W1c = W1[1324:2348]
    W1d = W1[2348:]
    out, feat = _tc_mlp(d_vecs, p_embeddings, ec, go, W1a, W1b, W1c, W1d,
                        b1, g1, be1, W2, b2, g2, be2, W3, b3, g3, be3, W4, b4)
    return (out, feat)


# speedup vs baseline: 1.1671x; 1.1671x over previous
import jax
import jax.numpy as jnp
from jax import lax
from jax.experimental import pallas as pl
from jax.experimental.pallas import tpu as pltpu
from jax.experimental.pallas import tpu_sc as plsc

N = 10000
E = 160000
B = 1024
NS = 16
NC = 2
L = 16
TILES = NC * NS
SLOTS_T = B // TILES
E_TILE = E // NS
CH = 800
CH_DEG = 400
NCH_DEG = E_TILE // CH_DEG
NCH_ALL = E // CH
CAP = 3584
FLUSH_T = CAP - CH - 2 * L
DD = 1024
DP = 2812
DPH = 1408
NPAD = 10240
SEG = NPAD // NS

f32 = jnp.float32
i32 = jnp.int32


def _leaky(x):
    return jnp.where(x > 0, x, 0.01 * x)


def _bn(x, gamma, beta, eps=1e-5):
    m = jnp.mean(x, axis=0, keepdims=True)
    v = jnp.mean((x - m) * (x - m), axis=0, keepdims=True)
    return (x - m) / jnp.sqrt(v + eps) * gamma + beta


def _frsqrt(x):
    i = plsc.bitcast(x, i32)
    y = plsc.bitcast(jnp.int32(0x5F3759DF) - (i >> 1), f32)
    for _ in range(3):
        y = y * (1.5 - 0.5 * x * y * y)
    return y


def _sc_body(d_row, d_col, d_ew, d_idx, p_row, p_col, p_ew, p_idx, xd, xp,
             agg, repd, repp,
             idx_tab, rep_tab, dinv, degtmp,
             row_buf, col_buf, ew_buf,
             slot_list, row_list, norm_list, idx16a, idx16b,
             stage_d, stage_p, acc, deg_scr, deg_sum, sem):
    c = lax.axis_index("c")
    s = lax.axis_index("s")
    t = c * NS + s
    ii = lax.iota(i32, L)
    zf = jnp.zeros((L,), f32)
    zi = jnp.zeros((L,), i32)

    def make_dinv(col_hbm, ew_hbm):
        def zv(v, _):
            dinv[pl.ds(v * L, L)] = zf
            return _
        lax.fori_loop(0, NPAD // L, zv, None)
        def chunk(ch, _):
            base = s * E_TILE + ch * CH_DEG
            pltpu.sync_copy(col_hbm.at[pl.ds(base, CH_DEG)],
                            col_buf.at[pl.ds(0, CH_DEG)])
            pltpu.sync_copy(ew_hbm.at[pl.ds(base, CH_DEG)],
                            ew_buf.at[pl.ds(0, CH_DEG)])
            def eb(e, _):
                cv = col_buf[pl.ds(e * L, L)]
                ev = ew_buf[pl.ds(e * L, L)]
                def db(rem):
                    plsc.store_scatter(rep_tab, [cv], ii, mask=rem)
                    win = (plsc.load_gather(rep_tab, [cv]) == ii) & rem
                    plsc.addupdate_scatter(dinv, [cv], ev, mask=win)
                    return rem & jnp.logical_not(win)
                lax.while_loop(jnp.any, db, cv == cv)
                return _
            lax.fori_loop(0, CH_DEG // L, eb, None)
            return _
        lax.fori_loop(0, NCH_DEG, chunk, None)
        def wr(k, _):
            pltpu.sync_copy(dinv.at[pl.ds(k * SEG, SEG)],
                            deg_scr.at[pl.ds(s * NPAD + k * SEG, SEG)])
            return _
        lax.fori_loop(0, NS, wr, None)
        plsc.subcore_barrier()
        def zv2(v, _):
            dinv[pl.ds(v * L, L)] = zf
            return _
        lax.fori_loop(0, SEG // L, zv2, None)
        def tsum(tt, _):
            pltpu.sync_copy(deg_scr.at[pl.ds(tt * NPAD + s * SEG, SEG)],
                            degtmp.at[pl.ds(0, SEG)])
            def vadd(v, _):
                dinv[pl.ds(v * L, L)] = (dinv[pl.ds(v * L, L)]
                                         + degtmp[pl.ds(v * L, L)])
                return _
            lax.fori_loop(0, SEG // L, vadd, None, unroll=8)
            return _
        lax.fori_loop(0, NS, tsum, None)
        pltpu.sync_copy(dinv.at[pl.ds(0, SEG)],
                        deg_sum.at[pl.ds(s * SEG, SEG)])
        plsc.subcore_barrier()
        def rd(k, _):
            pltpu.sync_copy(deg_sum.at[pl.ds(k * SEG, SEG)],
                            dinv.at[pl.ds(k * SEG, SEG)])
            return _
        lax.fori_loop(0, NS, rd, None)
        def nb(i, _):
            x = dinv[pl.ds(i * L, L)]
            dinv[pl.ds(i * L, L)] = _frsqrt(x + 1.0)
            return _
        lax.fori_loop(0, N // L, nb, None)

    def build_rep(idx_hbm):
        def ld(k, _):
            pltpu.sync_copy(idx_hbm.at[pl.ds(k * 512, 512)],
                            idx_tab.at[pl.ds(k * 512, 512)])
            return _
        lax.fori_loop(0, B // 512, ld, None)
        def zr(v, _):
            rep_tab[pl.ds(v * L, L)] = zi
            return _
        lax.fori_loop(0, N // L, zr, None)
        def br(j, _):
            cv = idx_tab[pl.ds(j * L, L)]
            plsc.store_scatter(rep_tab, [cv], ii + j * L)
            return _
        lax.fori_loop(0, B // L, br, None)

    def edge_flush(off, gather_fn, W, stage):
        ng = (off + L - 1) // L
        def g(k, _):
            idx16a[...] = row_list[pl.ds(k * L, L)]
            slotv = slot_list[pl.ds(k * L, L)] - t * SLOTS_T
            nv = norm_list[pl.ds(k * L, L)]
            gather_fn(idx16a, stage)
            for r in range(L):
                nsc = jnp.sum(jnp.where(ii == r, nv, 0.0))
                sl = jnp.sum(jnp.where(ii == r, slotv, 0))
                def fb(f, _):
                    acc[sl, pl.ds(f * L, L)] = (
                        acc[sl, pl.ds(f * L, L)]
                        + stage[r, pl.ds(f * L, L)] * nsc)
                    return _
                lax.fori_loop(0, W // L, fb, None, unroll=4)
            return _
        lax.fori_loop(0, ng, g, None)

    def pad_lists(off):
        slot_list[pl.ds(off, L)] = zi + (t * SLOTS_T + SLOTS_T)
        row_list[pl.ds(off, L)] = zi
        norm_list[pl.ds(off, L)] = zf

    def run_pass(row_hbm, col_hbm, ew_hbm, gather_fn, store_fn, W, stage):
        for rr in range(SLOTS_T + 1):
            def za(f, _):
                acc[rr, pl.ds(f * L, L)] = zf
                return _
            lax.fori_loop(0, W // L, za, None, unroll=4)

        def edge_scan(off):
            def vb(v, off):
                cv = col_buf[pl.ds(v * L, L)]
                rv = row_buf[pl.ds(v * L, L)]
                ev = ew_buf[pl.ds(v * L, L)]
                sv = plsc.load_gather(rep_tab, [cv])
                chk = plsc.load_gather(idx_tab, [sv])
                valid = (chk == cv) & ((sv >> 5) == t)
                dr = plsc.load_gather(dinv, [rv])
                dc = plsc.load_gather(dinv, [cv])
                nv = dr * ev * dc
                plsc.store_compressed(slot_list.at[pl.ds(off, L)], sv,
                                      mask=valid)
                plsc.store_compressed(row_list.at[pl.ds(off, L)], rv,
                                      mask=valid)
                plsc.store_compressed(norm_list.at[pl.ds(off, L)], nv,
                                      mask=valid)
                return off + jnp.sum(jnp.where(valid, 1, 0))
            return lax.fori_loop(0, CH // L, vb, off)

        def self_scan(off):
            def jb(j, off):
                bv = ii + j * L
                cv = idx_tab[pl.ds(j * L, L)]
                rv = plsc.load_gather(rep_tab, [cv])
                valid = (rv == bv) & ((bv >> 5) == t)
                dc = plsc.load_gather(dinv, [cv])
                nv = dc * dc
                plsc.store_compressed(slot_list.at[pl.ds(off, L)], bv,
                                      mask=valid)
                plsc.store_compressed(row_list.at[pl.ds(off, L)], cv,
                                      mask=valid)
                plsc.store_compressed(norm_list.at[pl.ds(off, L)], nv,
                                      mask=valid)
                return off + jnp.sum(jnp.where(valid, 1, 0))
            return lax.fori_loop(0, B // L, jb, off)

        def chunk(ch, off):
            is_tail = ch == NCH_ALL
            @pl.when(jnp.logical_not(is_tail))
            def _():
                base = ch * CH
                pltpu.sync_copy(row_hbm.at[pl.ds(base, CH)],
                                row_buf.at[pl.ds(0, CH)])
                pltpu.sync_copy(col_hbm.at[pl.ds(base, CH)],
                                col_buf.at[pl.ds(0, CH)])
                pltpu.sync_copy(ew_hbm.at[pl.ds(base, CH)],
                                ew_buf.at[pl.ds(0, CH)])
            off = lax.cond(is_tail, self_scan, edge_scan, off)
            def do_flush(o):
                pad_lists(o)
                edge_flush(o, gather_fn, W, stage)
                return jnp.int32(0)
            return lax.cond((off >= FLUSH_T) | is_tail, do_flush,
                            lambda o: o, off)
        lax.fori_loop(0, NCH_ALL + 1, chunk, jnp.int32(0))

        for g in range(SLOTS_T // L):
            for r in range(L):
                def mf(f, _):
                    stage[r, pl.ds(f * L, L)] = acc[g * L + r, pl.ds(f * L, L)]
                    return _
                lax.fori_loop(0, W // L, mf, None, unroll=4)
            store_fn(g, stage)

    def export_rep(rep_out):
        @pl.when(c == 0)
        def _():
            def jb(j, _):
                cv = idx_tab[pl.ds(s * (B // NS) + j * L, L)]
                rv = plsc.load_gather(rep_tab, [cv])
                row_buf[pl.ds(j * L, L)] = rv
                return _
            lax.fori_loop(0, (B // NS) // L, jb, None)
            pltpu.sync_copy(row_buf.at[pl.ds(0, B // NS)],
                            rep_out.at[pl.ds(s * (B // NS), B // NS)])

    make_dinv(d_col, d_ew)
    build_rep(d_idx)
    def d_gather(idx_ref, stage):
        pltpu.async_copy(xd.at[idx_ref], stage, sem).wait()
    def d_store(g, stage):
        row0 = pl.multiple_of(t * SLOTS_T + g * L, 8)
        pltpu.sync_copy(stage, agg.at[pl.ds(row0, L), pl.ds(0, DD)])
    run_pass(d_row, d_col, d_ew, d_gather, d_store, DD, stage_d)
    export_rep(repd)

    plsc.subcore_barrier()
    make_dinv(p_col, p_ew)
    build_rep(p_idx)
    def ph(h, _):
        col0 = pl.multiple_of(h * DPH, 128)
        def p_gather(idx_ref, stage):
            pltpu.async_copy(xp.at[idx_ref, pl.ds(col0, DPH)], stage,
                             sem).wait()
        def p_store(g, stage):
            row0 = pl.multiple_of(t * SLOTS_T + g * L, 8)
            colb = pl.multiple_of(DD + col0, 128)
            pltpu.sync_copy(stage, agg.at[pl.ds(row0, L), pl.ds(colb, DPH)])
        run_pass(p_row, p_col, p_ew, p_gather, p_store, DPH, stage_p)
        return _
    lax.fori_loop(0, 2, ph, None)
    export_rep(repp)


def _sc_aggregate(d_row, d_col, d_ew, d_idx, p_row, p_col, p_ew, p_idx, xd, xp):
    mesh = plsc.VectorSubcoreMesh(core_axis_name="c", subcore_axis_name="s")
    fn = pl.kernel(
        _sc_body,
        out_type=(
            jax.ShapeDtypeStruct((B, DD + 2 * DPH), f32),
            jax.ShapeDtypeStruct((B,), i32),
            jax.ShapeDtypeStruct((B,), i32),
        ),
        mesh=mesh,
        compiler_params=pltpu.CompilerParams(needs_layout_passes=False),
        scratch_types=[
            pltpu.VMEM((B,), i32),
            pltpu.VMEM((N,), i32),
            pltpu.VMEM((NPAD,), f32),
            pltpu.VMEM((SEG,), f32),
            pltpu.VMEM((CH + L,), i32),
            pltpu.VMEM((CH + L,), i32),
            pltpu.VMEM((CH + L,), f32),
            pltpu.VMEM((CAP,), i32),
            pltpu.VMEM((CAP,), i32),
            pltpu.VMEM((CAP,), f32),
            pltpu.VMEM((L,), i32),
            pltpu.VMEM((L,), i32),
            pltpu.VMEM((L, DD), f32),
            pltpu.VMEM((L, DPH), f32),
            pltpu.VMEM((SLOTS_T + 1, DPH), f32),
            pltpu.HBM((NS * NPAD,), f32),
            pltpu.HBM((NPAD,), f32),
            pltpu.SemaphoreType.DMA,
        ],
    )
    return fn(d_row, d_col, d_ew, d_idx, p_row, p_col, p_ew, p_idx, xd, xp)


def _tc_encode(agg, repd, repp, Wd, bd, Wp, bp):
    def body(ag_ref, rd_ref, rp_ref, wd_ref, bd_ref, wp_ref, bp_ref,
             ec_ref, go_ref):
        iota2 = lax.broadcasted_iota(i32, (B, B), 1)
        pd = (rd_ref[...] == iota2).astype(f32)
        pp = (rp_ref[...] == iota2).astype(f32)
        ag = ag_ref[...]
        rd = jnp.dot(pd, ag[:, :DD], preferred_element_type=f32)
        rp = jnp.dot(pp, ag[:, DD:DD + DP], preferred_element_type=f32)
        ec = jnp.dot(rd, wd_ref[...], preferred_element_type=f32)
        ec_ref[...] = _leaky(ec + bd_ref[...])
        go = jnp.dot(rp, wp_ref[...], preferred_element_type=f32)
        go_ref[...] = _leaky(go + bp_ref[...])
    return pl.pallas_call(
        body,
        compiler_params=pltpu.CompilerParams(
            vmem_limit_bytes=100 * 1024 * 1024),
        out_shape=(
            jax.ShapeDtypeStruct((B, 1024), f32),
            jax.ShapeDtypeStruct((B, 1024), f32),
        ),
    )(agg, repd.reshape(B, 1), repp.reshape(B, 1),
      Wd, bd.reshape(1, -1), Wp, bp.reshape(1, -1))


def _tc_mlp(dv, pe, ec, go, W1a, W1b, W1c, W1d, b1, g1, be1,
            W2, b2, g2, be2, W3, b3, g3, be3, W4, b4):
    def body(dv_ref, pe_ref, ec_ref, go_ref, w1a_ref, w1b_ref, w1c_ref,
             w1d_ref, b1_ref, g1_ref, be1_ref, w2_ref, b2_ref, g2_ref,
             be2_ref, w3_ref, b3_ref, g3_ref, be3_ref, w4_ref, b4_ref,
             out_ref, feat_ref):
        h = (jnp.dot(dv_ref[...], w1a_ref[...], preferred_element_type=f32)
             + jnp.dot(pe_ref[...], w1b_ref[...], preferred_element_type=f32)
             + jnp.dot(ec_ref[...], w1c_ref[...], preferred_element_type=f32)
             + jnp.dot(go_ref[...], w1d_ref[...], preferred_element_type=f32)
             + b1_ref[...])
        h = _leaky(_bn(h, g1_ref[...], be1_ref[...]))
        feat = _leaky(_bn(
            jnp.dot(h, w2_ref[...], preferred_element_type=f32) + b2_ref[...],
            g2_ref[...], be2_ref[...]))
        feat_ref[...] = feat
        z = _bn(_leaky(
            jnp.dot(feat, w3_ref[...], preferred_element_type=f32) + b3_ref[...]),
            g3_ref[...], be3_ref[...])
        out_ref[...] = jnp.dot(z, w4_ref[...], preferred_element_type=f32) + b4_ref[...]
    r2 = lambda a: a.reshape(1, -1)
    return pl.pallas_call(
        body,
        compiler_params=pltpu.CompilerParams(
            vmem_limit_bytes=100 * 1024 * 1024),
        out_shape=(
            jax.ShapeDtypeStruct((B, 1), f32),
            jax.ShapeDtypeStruct((B, 1024), f32),
        ),
    )(dv, pe, ec, go, W1a, W1b, W1c, W1d, r2(b1), r2(g1), r2(be1),
      W2, r2(b2), r2(g2), r2(be2), W3, r2(b3), r2(g3), r2(be3), W4, r2(b4))


def kernel(d_index, p_index, d_vecs, p_embeddings, y, d_ecfps, d_edge_index,
           d_edge_weight, p_gos, p_edge_index, p_edge_weight, Wd, bd, Wp, bp,
           W1, b1, g1, be1, W2, b2, g2, be2, W3, b3, g3, be3, W4, b4):
    d_row = d_edge_index[0].astype(i32)
    d_col = d_edge_index[1].astype(i32)
    p_row = p_edge_index[0].astype(i32)
    p_col = p_edge_index[1].astype(i32)
    d_idx = d_index.astype(i32)
    p_idx = p_index.astype(i32)

    xp_pad = jnp.pad(p_gos, ((0, 0), (0, 2 * DPH - DP)))
    agg, repd, repp = _sc_aggregate(
        d_row, d_col, d_edge_weight, d_idx,
        p_row, p_col, p_edge_weight, p_idx, d_ecfps, xp_pad)
    ec, go = _tc_encode(agg, repd, repp, Wd, bd, Wp, bp)
    W1a = W1[:300]
    W1b = W1[300:1324]
    W1c = W1[1324:2348]
    W1d = W1[2348:]
    out, feat = _tc_mlp(d_vecs, p_embeddings, ec, go, W1a, W1b, W1c, W1d,
                        b1, g1, be1, W2, b2, g2, be2, W3, b3, g3, be3, W4, b4)
    return (out, feat)
